# Initial kernel scaffold; baseline (speedup 1.0000x reference)
import jax
import jax.numpy as jnp
from jax.experimental import pallas as pl


def kernel(x, edge_index, edge_type, W):
    raise NotImplementedError("write your pallas kernel here")


# speedup vs baseline: 5.9757x; 5.9757x over previous
import functools

import jax
import jax.numpy as jnp
from jax import lax
from jax.experimental import pallas as pl
from jax.experimental.pallas import tpu as pltpu
from jax.experimental.pallas import tpu_sc as plsc

NC = 2
NS = 16
LANES = 16


def _matmul_y(x, w_all, n, remb):
    bn = 1000
    grid = (n // bn,)

    def body(x_ref, w_ref, o_ref):
        o_ref[...] = jnp.dot(x_ref[...], w_ref[...],
                             preferred_element_type=jnp.float32)

    emb = x.shape[1]
    return pl.pallas_call(
        body,
        grid=grid,
        in_specs=[
            pl.BlockSpec((bn, emb), lambda i: (i, 0)),
            pl.BlockSpec((emb, remb), lambda i: (0, 0)),
        ],
        out_specs=pl.BlockSpec((bn, remb), lambda i: (i, 0)),
        out_shape=jax.ShapeDtypeStruct((n, remb), jnp.float32),
    )(x, w_all)


def _add_relu(parts, n, emb):
    bn = 1000
    grid = (n // bn,)

    def body(p_ref, o_ref):
        o_ref[...] = jnp.maximum(p_ref[0] + p_ref[1], 0.0)

    return pl.pallas_call(
        body,
        grid=grid,
        in_specs=[pl.BlockSpec((NC, bn, emb), lambda i: (0, i, 0))],
        out_specs=pl.BlockSpec((bn, emb), lambda i: (i, 0)),
        out_shape=jax.ShapeDtypeStruct((n, emb), jnp.float32),
    )(parts)


def _sc_aggregate(y, esrc, edst, etype, n, r, emb, e):
    bins = n * r
    epw = e // (NC * NS)
    eps = e // NS
    ehalf = eps // 2
    rps = n // NS
    bps = bins // NS
    zb = ((bps + LANES - 1) // LANES) * LANES
    nchunks = epw // LANES
    chalf = ehalf // LANES
    depth = 16
    zrows = 640

    mesh = plsc.VectorSubcoreMesh(core_axis_name="c", subcore_axis_name="s")

    @functools.partial(
        pl.kernel,
        out_type=jax.ShapeDtypeStruct((NC, n, emb), jnp.float32),
        mesh=mesh,
        compiler_params=pltpu.CompilerParams(needs_layout_passes=False),
        scratch_types=[
            pltpu.VMEM((epw,), jnp.int32),
            pltpu.VMEM((epw,), jnp.int32),
            pltpu.VMEM((epw,), jnp.int32),
            pltpu.VMEM((zb,), jnp.float32),
            pltpu.VMEM((LANES, emb), jnp.float32),
            pltpu.VMEM((LANES, emb), jnp.float32),
            pltpu.VMEM((LANES,), jnp.float32),
            pltpu.VMEM((LANES,), jnp.float32),
            pltpu.VMEM_SHARED((bins,), jnp.float32),
            pltpu.VMEM_SHARED((n, emb), jnp.float32),
            pltpu.SemaphoreType.DMA,
            pltpu.SemaphoreType.DMA,
            pltpu.SemaphoreType.DMA,
            pltpu.SemaphoreType.DMA,
            pltpu.SemaphoreType.DMA,
            pltpu.SemaphoreType.DMA,
        ],
    )
    def sc_kernel(y_hbm, esrc_hbm, edst_hbm, et_hbm, out_hbm,
                  src_b, dst_b, rel_b, zbuf, rbuf_a, rbuf_b, ones_b, sbuf,
                  cnt_sp, acc_sp, sem_ga, sem_gb, sem_sa, sem_sb, sem_c,
                  sem_cv):
        c = lax.axis_index("c")
        s = lax.axis_index("s")
        wid = s * NC + c
        zero16 = jnp.zeros((LANES,), jnp.float32)

        for ee in range(LANES):
            for k in range(emb // LANES):
                rbuf_a[ee, pl.ds(k * LANES, LANES)] = zero16
        ones_b[...] = jnp.ones((LANES,), jnp.float32)
        zbase = s * (rps - 1)
        for j in range(zrows // LANES):
            pltpu.sync_copy(rbuf_a,
                            acc_sp.at[pl.ds(zbase + j * LANES, LANES)])
        for j in range(zb // LANES):
            zbuf[pl.ds(j * LANES, LANES)] = zero16
        pltpu.sync_copy(zbuf.at[pl.ds(0, bps)],
                        cnt_sp.at[pl.ds(s * bps, bps)])
        plsc.subcore_barrier()

        def drain_count():
            pltpu.make_async_copy(et_hbm.at[pl.ds(0, LANES)], ones_b,
                                  sem_c).wait()

        for h in range(2):
            base_e = s * eps + h * ehalf
            pltpu.sync_copy(esrc_hbm.at[pl.ds(base_e, ehalf)],
                            src_b.at[pl.ds(0, ehalf)])
            pltpu.sync_copy(et_hbm.at[pl.ds(base_e, ehalf)],
                            rel_b.at[pl.ds(0, ehalf)])

            def cbody(j, carry):
                src16 = src_b[pl.ds(j * LANES, LANES)]
                rel16 = rel_b[pl.ds(j * LANES, LANES)]
                fr16 = src16 + n * rel16
                pltpu.async_copy(ones_b, cnt_sp.at[fr16], sem_c, add=True)
                pl.when(j >= depth)(drain_count)
                return carry

            lax.fori_loop(0, chalf, cbody, None)
            for _ in range(depth):
                drain_count()
        plsc.subcore_barrier()

        base_w = wid * epw
        pltpu.sync_copy(esrc_hbm.at[pl.ds(base_w, epw)], src_b)
        pltpu.sync_copy(edst_hbm.at[pl.ds(base_w, epw)], dst_b)
        pltpu.sync_copy(et_hbm.at[pl.ds(base_w, epw)], rel_b)

        def handle(cidx, rbuf, sem_g, sem_s, first):
            if not first:
                pltpu.make_async_copy(y_hbm.at[pl.ds(0, LANES)], rbuf,
                                      sem_s).wait()
            base = cidx * LANES
            src16 = src_b[pl.ds(base, LANES)]
            dst16 = dst_b[pl.ds(base, LANES)]
            rel16 = rel_b[pl.ds(base, LANES)]
            g16 = dst16 * r + rel16
            gd = pltpu.async_copy(y_hbm.at[g16], rbuf, sem_g)
            fr16 = src16 + n * rel16
            cd = pltpu.async_copy(cnt_sp.at[fr16], sbuf, sem_cv)
            cd.wait()
            sbuf[...] = jnp.float32(1.0) / sbuf[...]
            gd.wait()
            sv = sbuf[...]
            for ee in range(LANES):
                spl = jnp.full((LANES,), sv[ee], jnp.float32)
                for k in range(emb // LANES):
                    sl = pl.ds(k * LANES, LANES)
                    rbuf[ee, sl] = rbuf[ee, sl] * spl
            pltpu.async_copy(rbuf, acc_sp.at[src16], sem_s, add=True)

        handle(0, rbuf_a, sem_ga, sem_sa, True)
        handle(1, rbuf_b, sem_gb, sem_sb, True)

        def mbody(j, carry):
            handle(2 * j, rbuf_a, sem_ga, sem_sa, False)
            handle(2 * j + 1, rbuf_b, sem_gb, sem_sb, False)
            return carry

        lax.fori_loop(1, nchunks // 2, mbody, None)
        handle(nchunks - 1, rbuf_a, sem_ga, sem_sa, False)
        pltpu.make_async_copy(y_hbm.at[pl.ds(0, LANES)], rbuf_a, sem_sa).wait()
        pltpu.make_async_copy(y_hbm.at[pl.ds(0, LANES)], rbuf_b, sem_sb).wait()
        plsc.subcore_barrier()

        pltpu.sync_copy(acc_sp.at[pl.ds(zbase, zrows)],
                        out_hbm.at[c, pl.ds(zbase, zrows)])

    return sc_kernel(y, esrc, edst, etype)


def kernel(x, edge_index, edge_type, W):
    n, emb = x.shape
    r = W.shape[0]
    e = edge_type.shape[0]
    w_all = jnp.transpose(W, (2, 0, 1)).reshape(emb, r * emb)
    y = _matmul_y(x, w_all, n, r * emb)
    y = y.reshape(n * r, emb)
    parts = _sc_aggregate(y, edge_index[0], edge_index[1], edge_type,
                          n, r, emb, e)
    return _add_relu(parts, n, emb)


# speedup vs baseline: 11.4704x; 1.9195x over previous
import functools

import jax
import jax.numpy as jnp
from jax import lax
from jax.experimental import pallas as pl
from jax.experimental.pallas import tpu as pltpu
from jax.experimental.pallas import tpu_sc as plsc

NC = 2
NS = 16
LANES = 16


def _matmul_y(x, w_all, n, remb):
    bn = 1000
    grid = (n // bn,)

    def body(x_ref, w_ref, o_ref):
        o_ref[...] = jnp.dot(x_ref[...], w_ref[...],
                             preferred_element_type=jnp.float32)

    emb = x.shape[1]
    return pl.pallas_call(
        body,
        grid=grid,
        in_specs=[
            pl.BlockSpec((bn, emb), lambda i: (i, 0)),
            pl.BlockSpec((emb, remb), lambda i: (0, 0)),
        ],
        out_specs=pl.BlockSpec((bn, remb), lambda i: (i, 0)),
        out_shape=jax.ShapeDtypeStruct((n, remb), jnp.float32),
    )(x, w_all)


def _add_relu(parts, n, emb):
    bn = 1000
    grid = (n // bn,)

    def body(p_ref, o_ref):
        o_ref[...] = jnp.maximum(p_ref[0] + p_ref[1], 0.0)

    return pl.pallas_call(
        body,
        grid=grid,
        in_specs=[pl.BlockSpec((NC, bn, emb), lambda i: (0, i, 0))],
        out_specs=pl.BlockSpec((bn, emb), lambda i: (i, 0)),
        out_shape=jax.ShapeDtypeStruct((n, emb), jnp.float32),
    )(parts)


def _sc_aggregate(y, esrc, edst, etype, n, r, emb, e):
    bins = n * r
    epw = e // (NC * NS)
    eps = e // NS
    ehalf = eps // 2
    rps = n // NS
    bps = bins // NS
    zb = ((bps + LANES - 1) // LANES) * LANES
    nchunks = epw // LANES
    chalf = ehalf // LANES
    depth = 16
    zrows = 640

    mesh = plsc.VectorSubcoreMesh(core_axis_name="c", subcore_axis_name="s")

    @functools.partial(
        pl.kernel,
        out_type=jax.ShapeDtypeStruct((NC, n, emb), jnp.float32),
        mesh=mesh,
        compiler_params=pltpu.CompilerParams(needs_layout_passes=False),
        scratch_types=[
            pltpu.VMEM((epw,), jnp.int32),
            pltpu.VMEM((epw,), jnp.int32),
            pltpu.VMEM((epw,), jnp.int32),
            pltpu.VMEM((zb,), jnp.float32),
            [pltpu.VMEM((LANES, emb), jnp.float32)] * 4,
            pltpu.VMEM((LANES,), jnp.float32),
            [pltpu.VMEM((LANES,), jnp.float32)] * 4,
            pltpu.VMEM_SHARED((bins,), jnp.float32),
            pltpu.VMEM_SHARED((n, emb), jnp.float32),
            [pltpu.SemaphoreType.DMA] * 4,
            [pltpu.SemaphoreType.DMA] * 4,
            [pltpu.SemaphoreType.DMA] * 4,
            pltpu.SemaphoreType.DMA,
        ],
    )
    def sc_kernel(y_hbm, esrc_hbm, edst_hbm, et_hbm, out_hbm,
                  src_b, dst_b, rel_b, zbuf, rbufs, ones_b, sbufs,
                  cnt_sp, acc_sp, sem_g, sem_s, sem_v, sem_c):
        c = lax.axis_index("c")
        s = lax.axis_index("s")
        wid = s * NC + c
        zero16 = jnp.zeros((LANES,), jnp.float32)

        for ee in range(LANES):
            for k in range(emb // LANES):
                rbufs[0][ee, pl.ds(k * LANES, LANES)] = zero16
        ones_b[...] = jnp.ones((LANES,), jnp.float32)
        zbase = s * (rps - 1)
        for j in range(zrows // LANES):
            pltpu.sync_copy(rbufs[0],
                            acc_sp.at[pl.ds(zbase + j * LANES, LANES)])
        for j in range(zb // LANES):
            zbuf[pl.ds(j * LANES, LANES)] = zero16
        pltpu.sync_copy(zbuf.at[pl.ds(0, bps)],
                        cnt_sp.at[pl.ds(s * bps, bps)])
        plsc.subcore_barrier()

        def drain_count():
            pltpu.make_async_copy(et_hbm.at[pl.ds(0, LANES)], ones_b,
                                  sem_c).wait()

        for h in range(2):
            base_e = s * eps + h * ehalf
            pltpu.sync_copy(esrc_hbm.at[pl.ds(base_e, ehalf)],
                            src_b.at[pl.ds(0, ehalf)])
            pltpu.sync_copy(et_hbm.at[pl.ds(base_e, ehalf)],
                            rel_b.at[pl.ds(0, ehalf)])

            def cbody(j, carry):
                src16 = src_b[pl.ds(j * LANES, LANES)]
                rel16 = rel_b[pl.ds(j * LANES, LANES)]
                fr16 = src16 + n * rel16
                pltpu.async_copy(ones_b, cnt_sp.at[fr16], sem_c, add=True)
                pl.when(j >= depth)(drain_count)
                return carry

            lax.fori_loop(0, chalf, cbody, None)
            for _ in range(depth):
                drain_count()
        plsc.subcore_barrier()

        base_w = wid * epw
        pltpu.sync_copy(esrc_hbm.at[pl.ds(base_w, epw)], src_b)
        pltpu.sync_copy(edst_hbm.at[pl.ds(base_w, epw)], dst_b)
        pltpu.sync_copy(et_hbm.at[pl.ds(base_w, epw)], rel_b)

        def fire(cidx, b, first):
            if not first:
                pltpu.make_async_copy(y_hbm.at[pl.ds(0, LANES)], rbufs[b],
                                      sem_s[b]).wait()
            base = cidx * LANES
            src16 = src_b[pl.ds(base, LANES)]
            dst16 = dst_b[pl.ds(base, LANES)]
            rel16 = rel_b[pl.ds(base, LANES)]
            g16 = dst16 * r + rel16
            pltpu.async_copy(y_hbm.at[g16], rbufs[b], sem_g[b])
            fr16 = src16 + n * rel16
            pltpu.async_copy(cnt_sp.at[fr16], sbufs[b], sem_v[b])

        def process(cidx, b):
            base = cidx * LANES
            src16 = src_b[pl.ds(base, LANES)]
            pltpu.make_async_copy(et_hbm.at[pl.ds(0, LANES)], sbufs[b],
                                  sem_v[b]).wait()
            sv = jnp.float32(1.0) / sbufs[b][...]
            pltpu.make_async_copy(y_hbm.at[pl.ds(0, LANES)], rbufs[b],
                                  sem_g[b]).wait()
            rbuf = rbufs[b]
            for ee in range(LANES):
                spl = jnp.full((LANES,), sv[ee], jnp.float32)
                for k in range(emb // LANES):
                    sl = pl.ds(k * LANES, LANES)
                    rbuf[ee, sl] = rbuf[ee, sl] * spl
            pltpu.async_copy(rbuf, acc_sp.at[src16], sem_s[b], add=True)

        for b in range(4):
            fire(b, b, True)

        def mbody(j, carry):
            c0 = 4 * j
            for b in range(4):
                process(c0 + b, b)
            for b in range(4):
                fire(c0 + 4 + b, b, False)
            return carry

        lax.fori_loop(0, (nchunks - 5) // 4, mbody, None)
        for b in range(4):
            process(620 + b, b)
        fire(nchunks - 1, 0, False)
        process(nchunks - 1, 0)
        for b in range(4):
            pltpu.make_async_copy(y_hbm.at[pl.ds(0, LANES)], rbufs[b],
                                  sem_s[b]).wait()
        plsc.subcore_barrier()

        pltpu.sync_copy(acc_sp.at[pl.ds(zbase, zrows)],
                        out_hbm.at[c, pl.ds(zbase, zrows)])

    return sc_kernel(y, esrc, edst, etype)


def kernel(x, edge_index, edge_type, W):
    n, emb = x.shape
    r = W.shape[0]
    e = edge_type.shape[0]
    w_all = jnp.transpose(W, (2, 0, 1)).reshape(emb, r * emb)
    y = _matmul_y(x, w_all, n, r * emb)
    y = y.reshape(n * r, emb)
    parts = _sc_aggregate(y, edge_index[0], edge_index[1], edge_type,
                          n, r, emb, e)
    return _add_relu(parts, n, emb)
